# baseline (speedup 1.0000x reference)
import functools

import jax
import jax.numpy as jnp
from jax import lax
from jax.experimental import pallas as pl
from jax.experimental.pallas import tpu as pltpu
from jax.experimental.pallas import tpu_sc as plsc

B = 16384
F_CAT = 26
F_REAL = 13
V = 100000
D = 32
F_TOT = F_CAT + F_REAL
FP = 40
DP = 128
NG = F_CAT * V // 8

NC = 2
NS = 16
NW = NC * NS
BPW = B // NW
GCH = 128
NCH = BPW // GCH
NK = F_CAT * NCH
BQ = B // GCH


def _sc_gather_body(uc_hbm, tab_hbm, t128_hbm, out_hbm,
                    idx2_v, idxs_v, t_v, gbuf_v, gsem, osem):
    c = lax.axis_index("c")
    s = lax.axis_index("s")
    wid = s * NC + c
    base = wid * BPW
    pltpu.sync_copy(uc_hbm.at[pl.ds(base, BPW)], idx2_v)
    pltpu.sync_copy(t128_hbm.at[pl.ds(0, F_CAT)], t_v)

    lanes = lax.broadcasted_iota(jnp.int32, (16,), 0)
    zeros = jnp.zeros((16,), jnp.int32)

    def ext_body(t, _):
        f = t // (BPW // 16)
        j = t % (BPW // 16)
        rows = lanes + j * 16
        fcol = zeros + f
        v = plsc.load_gather(idx2_v, [rows, fcol])
        idxs_v[f, pl.ds(j * 16, 16)] = v + f * V
        return 0

    lax.fori_loop(0, F_CAT * (BPW // 16), ext_body, 0)

    def issue_gather(k):
        f = k // NCH
        ch = k - f * NCH
        pltpu.async_copy(
            tab_hbm.at[idxs_v.at[f, pl.ds(ch * GCH, GCH)]],
            gbuf_v.at[k & 1],
            gsem,
        )

    def wait_gather(k):
        pltpu.make_async_copy(
            tab_hbm.at[idxs_v.at[0, pl.ds(0, GCH)]], gbuf_v.at[k & 1], gsem
        ).wait()

    def out_slice(k):
        f = k // NCH
        ch = k - f * NCH
        return out_hbm.at[pl.ds(base + ch * GCH, GCH), f, pl.ds(0, D)]

    def issue_out(k):
        pltpu.async_copy(gbuf_v.at[k & 1], out_slice(k), osem)

    def wait_out(k):
        pltpu.make_async_copy(gbuf_v.at[k & 1], out_slice(k), osem).wait()

    def typeadd(k):
        f = k // NCH
        par = k & 1
        t0 = t_v[f, pl.ds(0, 16)]
        t1 = t_v[f, pl.ds(16, 16)]

        def row_body(i, _):
            for u in range(8):
                r = i * 8 + u
                gbuf_v[par, r, pl.ds(0, 16)] = gbuf_v[par, r, pl.ds(0, 16)] + t0
                gbuf_v[par, r, pl.ds(16, 16)] = gbuf_v[par, r, pl.ds(16, 16)] + t1
            return 0

        lax.fori_loop(0, GCH // 8, row_body, 0)

    def pipe_body(k, _):
        @pl.when(k >= 2)
        def _():
            wait_out(k - 2)

        issue_gather(k)

        @pl.when(k >= 1)
        def _():
            wait_gather(k - 1)
            typeadd(k - 1)
            issue_out(k - 1)

        return 0

    lax.fori_loop(0, NK, pipe_body, 0)
    wait_gather(NK - 1)
    typeadd(NK - 1)
    issue_out(NK - 1)
    wait_out(NK - 2)
    wait_out(NK - 1)


_sc_gather = functools.partial(
    pl.kernel,
    out_type=jax.ShapeDtypeStruct((B, FP, DP), jnp.float32),
    mesh=plsc.VectorSubcoreMesh(core_axis_name="c", subcore_axis_name="s"),
    scratch_types=[
        pltpu.VMEM((BPW, F_CAT), jnp.int32),
        pltpu.VMEM((F_CAT, BPW), jnp.int32),
        pltpu.VMEM((F_CAT, DP), jnp.float32),
        pltpu.VMEM((2, GCH, D), jnp.float32),
        pltpu.SemaphoreType.DMA,
        pltpu.SemaphoreType.DMA,
    ],
    compiler_params=pltpu.CompilerParams(
        use_tc_tiling_on_sc=False, needs_layout_passes=False
    ),
)(_sc_gather_body)


def _tc_sweep_body(x_ref, w_ref, b_ref, g_ref, be_ref, t_ref, io_ref, out_ref):
    j = pl.program_id(1)
    io8 = io_ref[...]
    fields = lax.broadcasted_iota(jnp.int32, (1, 8, 1), 1) + 24 + j * 8
    catm = fields < F_CAT
    xf = x_ref[...]
    x = jnp.where(j == 0, xf[:, :8], xf[:, 8:])
    w = w_ref[...]
    b = b_ref[...]
    h = x[:, :, None] * w[None] + b[None]
    mu = jnp.sum(h, axis=-1, keepdims=True) * (1.0 / D)
    var = jnp.sum(h * h, axis=-1, keepdims=True) * (1.0 / D) - mu * mu
    hn = (h - mu) * lax.rsqrt(var + 1e-5)
    hn = hn * g_ref[...][None] + be_ref[...][None]
    hn = jnp.maximum(hn, 0.0)
    real8 = hn + t_ref[...][None]
    out_ref[...] = jnp.where(catm, io8, real8)


BBLK = 512


def _tc_sweep(xp, w128, b128, g128, be128, t128, combined):
    return pl.pallas_call(
        _tc_sweep_body,
        out_shape=jax.ShapeDtypeStruct((B, FP, DP), jnp.float32),
        grid=(B // BBLK, 2),
        in_specs=[
            pl.BlockSpec((BBLK, 16), lambda i, j: (i, 0)),
            pl.BlockSpec((8, DP), lambda i, j: (3 + j, 0)),
            pl.BlockSpec((8, DP), lambda i, j: (3 + j, 0)),
            pl.BlockSpec((8, DP), lambda i, j: (3 + j, 0)),
            pl.BlockSpec((8, DP), lambda i, j: (3 + j, 0)),
            pl.BlockSpec((8, DP), lambda i, j: (3 + j, 0)),
            pl.BlockSpec((BBLK, 8, DP), lambda i, j: (i, 3 + j, 0)),
        ],
        out_specs=pl.BlockSpec((BBLK, 8, DP), lambda i, j: (i, 3 + j, 0)),
        input_output_aliases={6: 0},
    )(xp, w128, b128, g128, be128, t128, combined)


def kernel(user_categoricals, user_reals, cat_tables, type_emb, real_w, real_b,
           ln_gamma, ln_beta):
    tab2 = cat_tables.reshape(F_CAT * V, D)
    t128 = jnp.pad(type_emb, ((0, FP - F_TOT), (0, DP - D)))
    combined = _sc_gather(user_categoricals, tab2, t128)
    xp = jnp.pad(user_reals, ((0, 0), (2, 1)))
    w128 = jnp.pad(real_w, ((F_CAT, 1), (0, DP - D)))
    b128 = jnp.pad(real_b, ((F_CAT, 1), (0, DP - D)))
    g128 = jnp.pad(ln_gamma, ((F_CAT, 1), (0, DP - D)))
    be128 = jnp.pad(ln_beta, ((F_CAT, 1), (0, DP - D)))
    swept = _tc_sweep(xp, w128, b128, g128, be128, t128, combined)
    return swept[:, :F_TOT, :D]


# speedup vs baseline: 1.0147x; 1.0147x over previous
import functools

import jax
import jax.numpy as jnp
from jax import lax
from jax.experimental import pallas as pl
from jax.experimental.pallas import tpu as pltpu
from jax.experimental.pallas import tpu_sc as plsc

B = 16384
F_CAT = 26
F_REAL = 13
V = 100000
D = 32
F_TOT = F_CAT + F_REAL
FP = 40
DP = 128

NC = 2
NS = 16
NW = NC * NS
BPW = B // NW
GCH = 128
NCH = BPW // GCH
NK = F_CAT * NCH


def _sc_gather_body(uc_hbm, tab_hbm, out_hbm, idx2_v, idxs_v, gbuf_v, gsem, osem):
    c = lax.axis_index("c")
    s = lax.axis_index("s")
    wid = s * NC + c
    base = wid * BPW
    pltpu.sync_copy(uc_hbm.at[pl.ds(base, BPW)], idx2_v)

    lanes = lax.broadcasted_iota(jnp.int32, (16,), 0)
    zeros = jnp.zeros((16,), jnp.int32)

    def ext_body(t, _):
        f = t // (BPW // 16)
        j = t % (BPW // 16)
        rows = lanes + j * 16
        fcol = zeros + f
        v = plsc.load_gather(idx2_v, [rows, fcol])
        idxs_v[f, pl.ds(j * 16, 16)] = v + f * V
        return 0

    lax.fori_loop(0, F_CAT * (BPW // 16), ext_body, 0)

    def issue_gather(k):
        f = k // NCH
        ch = k - f * NCH
        pltpu.async_copy(
            tab_hbm.at[idxs_v.at[f, pl.ds(ch * GCH, GCH)]],
            gbuf_v.at[k & 1],
            gsem,
        )

    def wait_gather(k):
        pltpu.make_async_copy(
            tab_hbm.at[idxs_v.at[0, pl.ds(0, GCH)]], gbuf_v.at[k & 1], gsem
        ).wait()

    def out_slice(k):
        f = k // NCH
        ch = k - f * NCH
        return out_hbm.at[pl.ds(base + ch * GCH, GCH), f, pl.ds(0, D)]

    def issue_out(k):
        pltpu.async_copy(gbuf_v.at[k & 1], out_slice(k), osem)

    def wait_out(k):
        pltpu.make_async_copy(gbuf_v.at[k & 1], out_slice(k), osem).wait()

    def pipe_body(k, _):
        @pl.when(k >= 2)
        def _():
            wait_out(k - 2)

        issue_gather(k)

        @pl.when(k >= 1)
        def _():
            wait_gather(k - 1)
            issue_out(k - 1)

        return 0

    lax.fori_loop(0, NK, pipe_body, 0)
    wait_gather(NK - 1)
    issue_out(NK - 1)
    wait_out(NK - 2)
    wait_out(NK - 1)


_sc_gather = functools.partial(
    pl.kernel,
    out_type=jax.ShapeDtypeStruct((B, FP, DP), jnp.float32),
    mesh=plsc.VectorSubcoreMesh(core_axis_name="c", subcore_axis_name="s"),
    scratch_types=[
        pltpu.VMEM((BPW, F_CAT), jnp.int32),
        pltpu.VMEM((F_CAT, BPW), jnp.int32),
        pltpu.VMEM((2, GCH, D), jnp.float32),
        pltpu.SemaphoreType.DMA,
        pltpu.SemaphoreType.DMA,
    ],
    compiler_params=pltpu.CompilerParams(
        use_tc_tiling_on_sc=False, needs_layout_passes=False
    ),
)(_sc_gather_body)


def _tc_sweep_body(x_ref, w_ref, b_ref, g_ref, be_ref, t_ref, io_ref, out_ref):
    io = io_ref[...]
    t128 = t_ref[...]
    cat = io[:, :F_CAT, :] + t128[None, :F_CAT, :]
    x = x_ref[...]
    w = w_ref[...]
    b = b_ref[...]
    h = x[:, :, None] * w[None] + b[None]
    mu = jnp.mean(h, axis=-1, keepdims=True)
    var = jnp.mean((h - mu) * (h - mu), axis=-1, keepdims=True)
    h = (h - mu) * lax.rsqrt(var + 1e-5)
    h = h * g_ref[...][None] + be_ref[...][None]
    h = jnp.maximum(h, 0.0)
    real = h + t128[None, F_CAT:F_TOT, :D]
    real128 = jnp.concatenate(
        [real, jnp.zeros((real.shape[0], F_REAL, DP - D), jnp.float32)], axis=-1)
    out_ref[...] = jnp.concatenate(
        [cat, real128, io[:, F_TOT:, :]], axis=1)


BBLK = 256


def _tc_sweep(ur, real_w, real_b, ln_gamma, ln_beta, t128, combined):
    return pl.pallas_call(
        _tc_sweep_body,
        out_shape=jax.ShapeDtypeStruct((B, FP, DP), jnp.float32),
        grid=(B // BBLK,),
        in_specs=[
            pl.BlockSpec((BBLK, F_REAL), lambda i: (i, 0)),
            pl.BlockSpec((F_REAL, D), lambda i: (0, 0)),
            pl.BlockSpec((F_REAL, D), lambda i: (0, 0)),
            pl.BlockSpec((F_REAL, D), lambda i: (0, 0)),
            pl.BlockSpec((F_REAL, D), lambda i: (0, 0)),
            pl.BlockSpec((FP, DP), lambda i: (0, 0)),
            pl.BlockSpec((BBLK, FP, DP), lambda i: (i, 0, 0)),
        ],
        out_specs=pl.BlockSpec((BBLK, FP, DP), lambda i: (i, 0, 0)),
        input_output_aliases={6: 0},
    )(ur, real_w, real_b, ln_gamma, ln_beta, t128, combined)


def kernel(user_categoricals, user_reals, cat_tables, type_emb, real_w, real_b,
           ln_gamma, ln_beta):
    tab2 = cat_tables.reshape(F_CAT * V, D)
    combined = _sc_gather(user_categoricals, tab2)
    t128 = jnp.pad(type_emb, ((0, FP - F_TOT), (0, DP - D)))
    swept = _tc_sweep(user_reals, real_w, real_b, ln_gamma, ln_beta,
                      t128, combined)
    return swept[:, :F_TOT, :D]
